# Initial kernel scaffold; baseline (speedup 1.0000x reference)
import jax
import jax.numpy as jnp
from jax.experimental import pallas as pl


def kernel(xINP, yINP):
    raise NotImplementedError("write your pallas kernel here")


# speedup vs baseline: 8527.2230x; 8527.2230x over previous
import jax
import jax.numpy as jnp
from jax import lax
from jax.experimental import pallas as pl
from jax.experimental.pallas import tpu as pltpu

_N = 1000
_NP = 1024
_INF = 1e30


def _match_kernel(x_ref, y_ref, out_ref, cost_ref):
    x = x_ref[...]
    y = y_ref[...]

    x2 = jnp.sum(x * x, axis=1, keepdims=True)
    y2 = jnp.sum(y * y, axis=1, keepdims=True)
    xy = lax.dot_general(x, y, (((1,), (1,)), ((), ())),
                         preferred_element_type=jnp.float32)
    cost = jnp.sqrt(jnp.maximum(x2 + y2.T - 2.0 * xy, 0.0))

    row_iota = lax.broadcasted_iota(jnp.int32, (_NP, _NP), 0)
    col_iota = lax.broadcasted_iota(jnp.int32, (_NP, _NP), 1)

    cost_ref[...] = jnp.where((row_iota < _N) & (col_iota < _N), cost, _INF)
    out_ref[...] = jnp.full((1, _NP), -1, jnp.int32)

    def cond(remaining):
        return remaining > 0

    def body(remaining):
        c = cost_ref[...]
        rmin = jnp.min(c, axis=1, keepdims=True)
        rarg = jnp.min(jnp.where(c == rmin, col_iota, _NP),
                       axis=1, keepdims=True)
        cmin = jnp.min(c, axis=0, keepdims=True)
        carg = jnp.min(jnp.where(c == cmin, row_iota, _NP),
                       axis=0, keepdims=True)
        dom = ((col_iota == rarg) & (row_iota == carg)
               & (c < _INF * 0.5))
        matched_r = jnp.any(dom, axis=1, keepdims=True)
        matched_c = jnp.any(dom, axis=0, keepdims=True)
        vals = jnp.max(jnp.where(dom, row_iota, -1), axis=0,
                       keepdims=True)
        out_ref[...] = jnp.where(matched_c, vals, out_ref[...])
        cost_ref[...] = jnp.where(matched_r | matched_c, _INF, c)
        return remaining - jnp.sum(dom.astype(jnp.int32))

    lax.while_loop(cond, body, jnp.int32(_N))


def kernel(xINP, yINP):
    x = jnp.zeros((_NP, xINP.shape[1]), jnp.float32).at[:_N].set(xINP)
    y = jnp.zeros((_NP, yINP.shape[1]), jnp.float32).at[:_N].set(yINP)
    c2r = pl.pallas_call(
        _match_kernel,
        out_shape=jax.ShapeDtypeStruct((1, _NP), jnp.int32),
        scratch_shapes=[pltpu.VMEM((_NP, _NP), jnp.float32)],
    )(x, y)
    return c2r[0, :_N]


# speedup vs baseline: 10895.0779x; 1.2777x over previous
import jax
import jax.numpy as jnp
from jax import lax
from jax.experimental import pallas as pl
from jax.experimental.pallas import tpu as pltpu

_N = 1000
_NP = 1024
_INF = 1e30
_STAGES = (1024, 512, 256, 128)


def _match_kernel(x_ref, y_ref, out_ref, cost_ref, ocid_ref, orid_ref,
                  c2rc_ref):
    x = x_ref[...]
    y = y_ref[...]
    x2 = jnp.sum(x * x, axis=1, keepdims=True)
    y2 = jnp.sum(y * y, axis=1, keepdims=True)
    xy = lax.dot_general(x, y, (((1,), (1,)), ((), ())),
                         preferred_element_type=jnp.float32)
    cost = jnp.sqrt(jnp.maximum(x2 + y2.T - 2.0 * xy, 0.0))

    row_iota = lax.broadcasted_iota(jnp.int32, (_NP, _NP), 0)
    col_iota = lax.broadcasted_iota(jnp.int32, (_NP, _NP), 1)
    cost_ref[...] = jnp.where((row_iota < _N) & (col_iota < _N), cost, _INF)
    iota_row = lax.broadcasted_iota(jnp.int32, (1, _NP), 1)
    ocid_ref[...] = iota_row
    orid_ref[...] = iota_row
    out_ref[...] = jnp.full((1, _NP), -1, jnp.int32)

    remaining = jnp.int32(_N)
    for i, size in enumerate(_STAGES):
        stop_at = _STAGES[i + 1] if i + 1 < len(_STAGES) else 0
        ri = lax.broadcasted_iota(jnp.int32, (size, size), 0)
        ci = lax.broadcasted_iota(jnp.int32, (size, size), 1)
        c2rc_ref[0:1, :size] = jnp.full((1, size), -1, jnp.int32)

        def cond(rem, _stop=stop_at):
            return rem > _stop

        def body(rem, _size=size, _ri=ri, _ci=ci):
            c = cost_ref[:_size, :_size]
            rmin = jnp.min(c, axis=1, keepdims=True)
            rarg = jnp.min(jnp.where(c == rmin, _ci, _size),
                           axis=1, keepdims=True)
            cmin = jnp.min(c, axis=0, keepdims=True)
            carg = jnp.min(jnp.where(c == cmin, _ri, _size),
                           axis=0, keepdims=True)
            dom = ((_ci == rarg) & (_ri == carg) & (c < _INF * 0.5))
            matched_r = jnp.any(dom, axis=1, keepdims=True)
            orid_col = orid_ref[0:1, :_size].T
            vals = jnp.max(jnp.where(dom, orid_col, -1), axis=0,
                           keepdims=True)
            matched_c = vals >= 0
            c2rc_ref[0:1, :_size] = jnp.where(matched_c, vals,
                                              c2rc_ref[0:1, :_size])
            cost_ref[:_size, :_size] = jnp.where(matched_r | matched_c,
                                                 _INF, c)
            return rem - jnp.sum(dom.astype(jnp.int32))

        remaining = lax.while_loop(cond, body, remaining)

        c2rc = c2rc_ref[0:1, :size]
        ocid = ocid_ref[0:1, :size]
        full_col = lax.broadcasted_iota(jnp.int32, (size, _NP), 1)
        upd = jnp.max(
            jnp.where((full_col == ocid.T) & (c2rc.T >= 0), c2rc.T, -1),
            axis=0, keepdims=True)
        out_ref[...] = jnp.maximum(out_ref[...], upd)

        if i + 1 < len(_STAGES):
            new = _STAGES[i + 1]
            c = cost_ref[:size, :size]
            free_r = jnp.min(c, axis=1, keepdims=True) < _INF * 0.5
            free_c = jnp.min(c, axis=0, keepdims=True) < _INF * 0.5
            fr = free_r.astype(jnp.float32)
            fc = free_c.astype(jnp.float32)
            lower = (ri >= ci).astype(jnp.float32)
            rank_r = lax.dot_general(lower, fr, (((1,), (0,)), ((), ())),
                                     precision=lax.Precision.HIGHEST)
            rank_c = lax.dot_general(fc, lower, (((1,), (1,)), ((), ())),
                                     precision=lax.Precision.HIGHEST)
            dst_r = rank_r.astype(jnp.int32) - 1
            dst_c = rank_c.astype(jnp.int32) - 1
            sri = lax.broadcasted_iota(jnp.int32, (new, size), 0)
            P = ((sri == dst_r.T) & free_r.T).astype(jnp.float32)
            Q = ((sri == dst_c) & free_c).astype(jnp.float32)
            pc = lax.dot_general(P, c, (((1,), (0,)), ((), ())),
                                 precision=lax.Precision.HIGHEST)
            cc = lax.dot_general(pc, Q, (((1,), (1,)), ((), ())),
                                 precision=lax.Precision.HIGHEST)
            nri = lax.broadcasted_iota(jnp.int32, (new, new), 0)
            nci = lax.broadcasted_iota(jnp.int32, (new, new), 1)
            nfree_r = jnp.sum(fr).astype(jnp.int32)
            nfree_c = jnp.sum(fc).astype(jnp.int32)
            cc = jnp.where((nri < nfree_r) & (nci < nfree_c), cc, _INF)
            ocid_f = ocid.astype(jnp.float32)
            orid_f = orid_ref[0:1, :size].astype(jnp.float32)
            new_ocid = lax.dot_general(ocid_f, Q, (((1,), (1,)), ((), ())),
                                       precision=lax.Precision.HIGHEST)
            new_orid = lax.dot_general(orid_f, P, (((1,), (1,)), ((), ())),
                                       precision=lax.Precision.HIGHEST)
            cost_ref[:new, :new] = cc
            ocid_ref[0:1, :new] = new_ocid.astype(jnp.int32)
            orid_ref[0:1, :new] = new_orid.astype(jnp.int32)


def kernel(xINP, yINP):
    x = jnp.zeros((_NP, xINP.shape[1]), jnp.float32).at[:_N].set(xINP)
    y = jnp.zeros((_NP, yINP.shape[1]), jnp.float32).at[:_N].set(yINP)
    c2r = pl.pallas_call(
        _match_kernel,
        out_shape=jax.ShapeDtypeStruct((1, _NP), jnp.int32),
        scratch_shapes=[
            pltpu.VMEM((_NP, _NP), jnp.float32),
            pltpu.VMEM((1, _NP), jnp.int32),
            pltpu.VMEM((1, _NP), jnp.int32),
            pltpu.VMEM((1, _NP), jnp.int32),
        ],
    )(x, y)
    return c2r[0, :_N]


# speedup vs baseline: 44568.3727x; 4.0907x over previous
import jax
import jax.numpy as jnp
from jax import lax
from jax.experimental import pallas as pl
from jax.experimental.pallas import tpu as pltpu

_N = 1000
_NP = 1024
_INF = 1e30
_STAGES = (1024, 512, 256, 128)


def _match_kernel(x_ref, y_ref, out_ref, cost_ref, ocid_ref, orid_ref,
                  c2rc_ref):
    x = x_ref[...]
    y = y_ref[...]
    x2 = jnp.sum(x * x, axis=1, keepdims=True)
    y2 = jnp.sum(y * y, axis=1, keepdims=True)
    xy = lax.dot_general(x, y, (((1,), (1,)), ((), ())),
                         preferred_element_type=jnp.float32)
    cost = jnp.sqrt(jnp.maximum(x2 + y2.T - 2.0 * xy, 0.0))

    row_iota = lax.broadcasted_iota(jnp.int32, (_NP, _NP), 0)
    col_iota = lax.broadcasted_iota(jnp.int32, (_NP, _NP), 1)
    cost_ref[...] = jnp.where((row_iota < _N) & (col_iota < _N), cost, _INF)
    iota_row = lax.broadcasted_iota(jnp.int32, (1, _NP), 1)
    ocid_ref[...] = iota_row
    orid_ref[...] = iota_row
    out_ref[...] = jnp.full((1, _NP), -1, jnp.int32)

    remaining = jnp.int32(_N)
    for i, size in enumerate(_STAGES):
        stop_at = _STAGES[i + 1] if i + 1 < len(_STAGES) else 0
        ri = lax.broadcasted_iota(jnp.int32, (size, size), 0)
        ci = lax.broadcasted_iota(jnp.int32, (size, size), 1)
        c2rc_ref[0:1, :size] = jnp.full((1, size), -1, jnp.int32)

        def cond(rem, _stop=stop_at):
            return rem > _stop

        def body(rem, _size=size, _ri=ri, _ci=ci):
            c = cost_ref[:_size, :_size]
            rmin = jnp.min(c, axis=1, keepdims=True)
            rarg = jnp.min(jnp.where(c == rmin, _ci, _size),
                           axis=1, keepdims=True)
            cmin = jnp.min(c, axis=0, keepdims=True)
            carg = jnp.min(jnp.where(c == cmin, _ri, _size),
                           axis=0, keepdims=True)
            dom = ((_ci == rarg) & (_ri == carg) & (c < _INF * 0.5))
            matched_r = jnp.any(dom, axis=1, keepdims=True)
            orid_col = orid_ref[0:1, :_size].T
            vals = jnp.max(jnp.where(dom, orid_col, -1), axis=0,
                           keepdims=True)
            matched_c = vals >= 0
            c2rc_ref[0:1, :_size] = jnp.where(matched_c, vals,
                                              c2rc_ref[0:1, :_size])
            cost_ref[:_size, :_size] = jnp.where(matched_r | matched_c,
                                                 _INF, c)
            return rem - jnp.sum(dom.astype(jnp.int32))

        remaining = lax.while_loop(cond, body, remaining)

        c2rc = c2rc_ref[0:1, :size]
        ocid = ocid_ref[0:1, :size]
        if i == 0:
            out_ref[...] = jnp.maximum(out_ref[...], c2rc)
        else:
            full_col = lax.broadcasted_iota(jnp.int32, (size, _NP), 1)
            upd = jnp.max(
                jnp.where((full_col == ocid.T) & (c2rc.T >= 0), c2rc.T, -1),
                axis=0, keepdims=True)
            out_ref[...] = jnp.maximum(out_ref[...], upd)

        if i + 1 < len(_STAGES):
            new = _STAGES[i + 1]
            c = cost_ref[:size, :size]
            free_r = jnp.min(c, axis=1, keepdims=True) < _INF * 0.5
            free_c = jnp.min(c, axis=0, keepdims=True) < _INF * 0.5
            fr = free_r.astype(jnp.float32)
            fc = free_c.astype(jnp.float32)
            lower = (ri >= ci).astype(jnp.bfloat16)
            rank_r = lax.dot_general(lower, fr.astype(jnp.bfloat16),
                                     (((1,), (0,)), ((), ())),
                                     preferred_element_type=jnp.float32)
            rank_c = lax.dot_general(fc.astype(jnp.bfloat16), lower,
                                     (((1,), (1,)), ((), ())),
                                     preferred_element_type=jnp.float32)
            dst_r = rank_r.astype(jnp.int32) - 1
            dst_c = rank_c.astype(jnp.int32) - 1
            sri = lax.broadcasted_iota(jnp.int32, (new, size), 0)
            P = ((sri == dst_r.T) & free_r.T).astype(jnp.float32)
            Q = ((sri == dst_c) & free_c).astype(jnp.float32)
            pc = lax.dot_general(P, c, (((1,), (0,)), ((), ())),
                                 precision=lax.Precision.HIGHEST)
            cc = lax.dot_general(pc, Q, (((1,), (1,)), ((), ())),
                                 precision=lax.Precision.HIGHEST)
            nri = lax.broadcasted_iota(jnp.int32, (new, new), 0)
            nci = lax.broadcasted_iota(jnp.int32, (new, new), 1)
            nfree_r = jnp.sum(fr).astype(jnp.int32)
            nfree_c = jnp.sum(fc).astype(jnp.int32)
            cc = jnp.where((nri < nfree_r) & (nci < nfree_c), cc, _INF)
            def _remap(ids, onehot):
                hi = (ids // 32 * 32).astype(jnp.bfloat16)
                lo = (ids % 32).astype(jnp.bfloat16)
                hl = jnp.concatenate([hi, lo], axis=0)
                nh = lax.dot_general(hl, onehot.astype(jnp.bfloat16),
                                     (((1,), (1,)), ((), ())),
                                     preferred_element_type=jnp.float32)
                return (nh[0:1, :] + nh[1:2, :]).astype(jnp.int32)

            cost_ref[:new, :new] = cc
            ocid_ref[0:1, :new] = _remap(ocid, Q)
            orid_ref[0:1, :new] = _remap(orid_ref[0:1, :size], P)


def kernel(xINP, yINP):
    x = jnp.zeros((_NP, xINP.shape[1]), jnp.float32).at[:_N].set(xINP)
    y = jnp.zeros((_NP, yINP.shape[1]), jnp.float32).at[:_N].set(yINP)
    c2r = pl.pallas_call(
        _match_kernel,
        out_shape=jax.ShapeDtypeStruct((1, _NP), jnp.int32),
        scratch_shapes=[
            pltpu.VMEM((_NP, _NP), jnp.float32),
            pltpu.VMEM((1, _NP), jnp.int32),
            pltpu.VMEM((1, _NP), jnp.int32),
            pltpu.VMEM((1, _NP), jnp.int32),
        ],
    )(x, y)
    return c2r[0, :_N]
